# baseline (speedup 1.0000x reference)
import functools

import jax
import jax.numpy as jnp
from jax import lax
from jax.experimental import pallas as pl
from jax.experimental.pallas import tpu as pltpu
from jax.experimental.pallas import tpu_sc as plsc

D, H, E, N = 1024, 1024, 8, 4096
T = 512
G = N // T + E
N_PAD = G * T

_NC, _NS = 2, 16
_NW = _NC * _NS
_CHUNK = 32


def _sc_scatter_rows(Dm, B, V):
  b_per_w = B // _NW
  n_chunks = b_per_w // _CHUNK
  assert b_per_w % _CHUNK == 0 and B % (8 * _NW) == 0
  mesh = plsc.VectorSubcoreMesh(core_axis_name="c", subcore_axis_name="s")

  @functools.partial(
      pl.kernel,
      mesh=mesh,
      out_type=jax.ShapeDtypeStruct((V, Dm), jnp.float32),
      scratch_types=[
          pltpu.VMEM((n_chunks, _CHUNK), jnp.int32),
          pltpu.VMEM((_CHUNK, Dm), jnp.float32),
          pltpu.VMEM((_CHUNK, Dm), jnp.float32),
          pltpu.VMEM((_CHUNK, Dm), jnp.float32),
          pltpu.SemaphoreType.DMA,
          pltpu.SemaphoreType.DMA,
          pltpu.SemaphoreType.DMA,
          pltpu.SemaphoreType.DMA,
          pltpu.SemaphoreType.DMA,
          pltpu.SemaphoreType.DMA,
      ],
  )
  def k(rows_hbm, idx_hbm, out_hbm, idx_v, b0, b1, b2, l0, l1, l2, s0, s1, s2):
    wid = lax.axis_index("s") * _NC + lax.axis_index("c")
    base = wid * b_per_w
    pltpu.sync_copy(idx_hbm.at[wid], idx_v)
    bufs, lsem, ssem = (b0, b1, b2), (l0, l1, l2), (s0, s1, s2)
    nb = len(bufs)
    loads = [None] * nb
    scats = [None] * nb

    def issue_load(c):
      loads[c % nb] = pltpu.async_copy(
          rows_hbm.at[pl.ds(base + c * _CHUNK, _CHUNK)],
          bufs[c % nb], lsem[c % nb])

    for c in range(min(nb, n_chunks)):
      issue_load(c)
    for c in range(n_chunks):
      b = c % nb
      loads[b].wait()
      scats[b] = pltpu.async_copy(bufs[b], out_hbm.at[idx_v.at[c]], ssem[b])
      if c + nb < n_chunks:
        scats[b].wait()
        scats[b] = None
        issue_load(c + nb)
    for p in scats:
      if p is not None:
        p.wait()

  return k


def _sc_gather_rows(V, Dm, B):
  b_per_w = B // _NW
  n_chunks = b_per_w // _CHUNK
  assert b_per_w % _CHUNK == 0 and B % (8 * _NW) == 0
  mesh = plsc.VectorSubcoreMesh(core_axis_name="c", subcore_axis_name="s")

  @functools.partial(
      pl.kernel,
      mesh=mesh,
      out_type=jax.ShapeDtypeStruct((B, Dm), jnp.float32),
      scratch_types=[
          pltpu.VMEM((b_per_w,), jnp.int32),
          pltpu.VMEM((_CHUNK, Dm), jnp.float32),
          pltpu.VMEM((_CHUNK, Dm), jnp.float32),
          pltpu.VMEM((_CHUNK, Dm), jnp.float32),
          pltpu.SemaphoreType.DMA,
          pltpu.SemaphoreType.DMA,
          pltpu.SemaphoreType.DMA,
          pltpu.SemaphoreType.DMA,
          pltpu.SemaphoreType.DMA,
          pltpu.SemaphoreType.DMA,
      ],
  )
  def k(table_hbm, idx_hbm, out_hbm, idx_v, b0, b1, b2, g0, g1, g2, w0, w1, w2):
    wid = lax.axis_index("s") * _NC + lax.axis_index("c")
    base = wid * b_per_w
    pltpu.sync_copy(idx_hbm.at[pl.ds(base, b_per_w)], idx_v)
    bufs, gsem, wsem = (b0, b1, b2), (g0, g1, g2), (w0, w1, w2)
    nb = len(bufs)
    gaths = [None] * nb
    writes = [None] * nb

    def issue_gather(c):
      gaths[c % nb] = pltpu.async_copy(
          table_hbm.at[idx_v.at[pl.ds(c * _CHUNK, _CHUNK)]],
          bufs[c % nb], gsem[c % nb])

    for c in range(min(nb, n_chunks)):
      issue_gather(c)
    for c in range(n_chunks):
      b = c % nb
      gaths[b].wait()
      writes[b] = pltpu.async_copy(
          bufs[b], out_hbm.at[pl.ds(base + c * _CHUNK, _CHUNK)], wsem[b])
      if c + nb < n_chunks:
        writes[b].wait()
        writes[b] = None
        issue_gather(c + nb)
    for p in writes:
      if p is not None:
        p.wait()

  return k


def _ffn_body(t_ref, v_ref, tr_ref, bi_ref, x_ref, w1_hbm, b1_ref, w2_hbm,
              b2_ref, o_ref, w1s, w2s, s1a, s1b, s2a, s2b):
  t = pl.program_id(0)
  e = t_ref[t]
  bi = bi_ref[t]
  t_n = jnp.minimum(t + 1, G - 1)
  e_n = t_ref[t_n]
  bi_n = bi_ref[t_n]

  @pl.when(t == 0)
  def _():
    pltpu.make_async_copy(w1_hbm.at[e], w1s.at[0], s1a).start()
    pltpu.make_async_copy(w2_hbm.at[e], w2s.at[0], s2a).start()

  @pl.when(tr_ref[t] == 1)
  def _():
    @pl.when(bi == 0)
    def _():
      pltpu.make_async_copy(w1_hbm.at[e], w1s.at[0], s1a).wait()
      pltpu.make_async_copy(w2_hbm.at[e], w2s.at[0], s2a).wait()

    @pl.when(bi == 1)
    def _():
      pltpu.make_async_copy(w1_hbm.at[e], w1s.at[1], s1b).wait()
      pltpu.make_async_copy(w2_hbm.at[e], w2s.at[1], s2b).wait()

  @pl.when((t + 1 < G) & (tr_ref[t_n] == 1))
  def _():
    @pl.when(bi_n == 0)
    def _():
      pltpu.make_async_copy(w1_hbm.at[e_n], w1s.at[0], s1a).start()
      pltpu.make_async_copy(w2_hbm.at[e_n], w2s.at[0], s2a).start()

    @pl.when(bi_n == 1)
    def _():
      pltpu.make_async_copy(w1_hbm.at[e_n], w1s.at[1], s1b).start()
      pltpu.make_async_copy(w2_hbm.at[e_n], w2s.at[1], s2b).start()

  @pl.when(v_ref[t] == 1)
  def _():
    xb = x_ref[...].astype(jnp.bfloat16)
    h = lax.dot_general(
        xb, w1s[bi].astype(jnp.bfloat16), (((1,), (1,)), ((), ())),
        preferred_element_type=jnp.float32)
    h = jnp.maximum(h + b1_ref[0], 0.0).astype(jnp.bfloat16)
    o_ref[...] = lax.dot_general(
        h, w2s[bi].astype(jnp.bfloat16), (((1,), (1,)), ((), ())),
        preferred_element_type=jnp.float32) + b2_ref[0]


def _grouped_ffn(t_ids, valid, trans, bufid, x_sorted, W1, b1, W2, b2):
  return pl.pallas_call(
      _ffn_body,
      grid_spec=pltpu.PrefetchScalarGridSpec(
          num_scalar_prefetch=4,
          grid=(G,),
          in_specs=[
              pl.BlockSpec((T, D), lambda t, s, v, tr, bi: (t, 0)),
              pl.BlockSpec(memory_space=pl.ANY),
              pl.BlockSpec((1, 1, H), lambda t, s, v, tr, bi: (s[t], 0, 0)),
              pl.BlockSpec(memory_space=pl.ANY),
              pl.BlockSpec((1, 1, H), lambda t, s, v, tr, bi: (s[t], 0, 0)),
          ],
          out_specs=pl.BlockSpec((T, H), lambda t, s, v, tr, bi: (t, 0)),
          scratch_shapes=[
              pltpu.VMEM((2, H, D), jnp.float32),
              pltpu.VMEM((2, H, H), jnp.float32),
              pltpu.SemaphoreType.DMA,
              pltpu.SemaphoreType.DMA,
              pltpu.SemaphoreType.DMA,
              pltpu.SemaphoreType.DMA,
          ],
      ),
      out_shape=jax.ShapeDtypeStruct((N_PAD, H), jnp.float32),
  )(t_ids, valid, trans, bufid, x_sorted, W1, b1, W2, b2)


def kernel(x, Wr, br, W1, b1, W2, b2):
  x_fp32 = x.astype(jnp.float32)
  routing_logits = x_fp32 @ Wr.T + br
  expert_ids = jnp.argmax(routing_logits, axis=1)

  onehot = (expert_ids[:, None] == jnp.arange(E, dtype=expert_ids.dtype)
            ).astype(jnp.int32)
  csum = jnp.cumsum(onehot, axis=0)
  counts = csum[-1]
  tiles_e = (counts + T - 1) // T
  tile_end = jnp.cumsum(tiles_e)
  row_start = (tile_end - tiles_e) * T
  slot = jnp.sum((csum + row_start[None, :] - 1) * onehot, axis=1)
  tile_idx = jnp.arange(G, dtype=jnp.int32)
  raw = jnp.sum((tile_idx[:, None] >= tile_end[None, :]).astype(jnp.int32),
                axis=1)
  valid = (tile_idx < tile_end[-1]).astype(jnp.int32)
  last_e = jnp.max(jnp.where(counts > 0, jnp.arange(E), 0)).astype(jnp.int32)
  t_ids = jnp.where(valid == 1, raw, last_e).astype(jnp.int32)
  prev = jnp.concatenate([jnp.full((1,), -1, jnp.int32), t_ids[:-1]])
  trans = (t_ids != prev).astype(jnp.int32)
  bufid = ((jnp.cumsum(trans) - 1) % 2).astype(jnp.int32)

  slot3 = slot.astype(jnp.int32).reshape(_NW, (N // _NW) // _CHUNK, _CHUNK)
  x_sorted = _sc_scatter_rows(D, N, N_PAD)(x, slot3)

  out_sorted = _grouped_ffn(t_ids, valid, trans, bufid, x_sorted, W1,
                            b1[:, None, :], W2, b2[:, None, :])

  outputs = _sc_gather_rows(N_PAD, H, N)(out_sorted, slot.astype(jnp.int32))
  return outputs, expert_ids


# speedup vs baseline: 1.0178x; 1.0178x over previous
import functools

import jax
import jax.numpy as jnp
from jax import lax
from jax.experimental import pallas as pl
from jax.experimental.pallas import tpu as pltpu
from jax.experimental.pallas import tpu_sc as plsc

D, H, E, N = 1024, 1024, 8, 4096
T = 512
G = N // T + E
N_PAD = G * T

_NC, _NS = 2, 16
_NW = _NC * _NS
_CHUNK = 32


def _sc_scatter_rows(Dm, B, V):
  b_per_w = B // _NW
  n_chunks = b_per_w // _CHUNK
  assert b_per_w % _CHUNK == 0 and B % (8 * _NW) == 0
  mesh = plsc.VectorSubcoreMesh(core_axis_name="c", subcore_axis_name="s")

  @functools.partial(
      pl.kernel,
      mesh=mesh,
      out_type=jax.ShapeDtypeStruct((V, Dm), jnp.float32),
      scratch_types=[
          pltpu.VMEM((n_chunks, _CHUNK), jnp.int32),
          pltpu.VMEM((_CHUNK, Dm), jnp.float32),
          pltpu.VMEM((_CHUNK, Dm), jnp.float32),
          pltpu.VMEM((_CHUNK, Dm), jnp.float32),
          pltpu.SemaphoreType.DMA,
          pltpu.SemaphoreType.DMA,
          pltpu.SemaphoreType.DMA,
          pltpu.SemaphoreType.DMA,
          pltpu.SemaphoreType.DMA,
          pltpu.SemaphoreType.DMA,
      ],
  )
  def k(rows_hbm, idx_hbm, out_hbm, idx_v, b0, b1, b2, l0, l1, l2, s0, s1, s2):
    wid = lax.axis_index("s") * _NC + lax.axis_index("c")
    base = wid * b_per_w
    pltpu.sync_copy(idx_hbm.at[wid], idx_v)
    bufs, lsem, ssem = (b0, b1, b2), (l0, l1, l2), (s0, s1, s2)
    nb = len(bufs)
    loads = [None] * nb
    scats = [None] * nb

    def issue_load(c):
      loads[c % nb] = pltpu.async_copy(
          rows_hbm.at[pl.ds(base + c * _CHUNK, _CHUNK)],
          bufs[c % nb], lsem[c % nb])

    for c in range(min(nb, n_chunks)):
      issue_load(c)
    for c in range(n_chunks):
      b = c % nb
      loads[b].wait()
      scats[b] = pltpu.async_copy(bufs[b], out_hbm.at[idx_v.at[c]], ssem[b])
      if c + nb < n_chunks:
        scats[b].wait()
        scats[b] = None
        issue_load(c + nb)
    for p in scats:
      if p is not None:
        p.wait()

  return k


def _sc_gather_rows(V, Dm, B):
  b_per_w = B // _NW
  n_chunks = b_per_w // _CHUNK
  assert b_per_w % _CHUNK == 0 and B % (8 * _NW) == 0
  mesh = plsc.VectorSubcoreMesh(core_axis_name="c", subcore_axis_name="s")

  @functools.partial(
      pl.kernel,
      mesh=mesh,
      out_type=jax.ShapeDtypeStruct((B, Dm), jnp.float32),
      scratch_types=[
          pltpu.VMEM((b_per_w,), jnp.int32),
          pltpu.VMEM((_CHUNK, Dm), jnp.float32),
          pltpu.VMEM((_CHUNK, Dm), jnp.float32),
          pltpu.VMEM((_CHUNK, Dm), jnp.float32),
          pltpu.SemaphoreType.DMA,
          pltpu.SemaphoreType.DMA,
          pltpu.SemaphoreType.DMA,
          pltpu.SemaphoreType.DMA,
          pltpu.SemaphoreType.DMA,
          pltpu.SemaphoreType.DMA,
      ],
  )
  def k(table_hbm, idx_hbm, out_hbm, idx_v, b0, b1, b2, g0, g1, g2, w0, w1, w2):
    wid = lax.axis_index("s") * _NC + lax.axis_index("c")
    base = wid * b_per_w
    pltpu.sync_copy(idx_hbm.at[pl.ds(base, b_per_w)], idx_v)
    bufs, gsem, wsem = (b0, b1, b2), (g0, g1, g2), (w0, w1, w2)
    nb = len(bufs)
    gaths = [None] * nb
    writes = [None] * nb

    def issue_gather(c):
      gaths[c % nb] = pltpu.async_copy(
          table_hbm.at[idx_v.at[pl.ds(c * _CHUNK, _CHUNK)]],
          bufs[c % nb], gsem[c % nb])

    for c in range(min(nb, n_chunks)):
      issue_gather(c)
    for c in range(n_chunks):
      b = c % nb
      gaths[b].wait()
      writes[b] = pltpu.async_copy(
          bufs[b], out_hbm.at[pl.ds(base + c * _CHUNK, _CHUNK)], wsem[b])
      if c + nb < n_chunks:
        writes[b].wait()
        writes[b] = None
        issue_gather(c + nb)
    for p in writes:
      if p is not None:
        p.wait()

  return k


def _ffn_body(t_ref, v_ref, x_ref, w1_ref, b1_ref, w2_ref, b2_ref, o_ref):
  del t_ref
  t = pl.program_id(0)

  @pl.when(v_ref[t] == 1)
  def _():
    xb = x_ref[...].astype(jnp.bfloat16)
    h = lax.dot_general(
        xb, w1_ref[0].astype(jnp.bfloat16), (((1,), (1,)), ((), ())),
        preferred_element_type=jnp.float32)
    h = jnp.maximum(h + b1_ref[0], 0.0).astype(jnp.bfloat16)
    o_ref[...] = lax.dot_general(
        h, w2_ref[0].astype(jnp.bfloat16), (((1,), (1,)), ((), ())),
        preferred_element_type=jnp.float32) + b2_ref[0]


def _grouped_ffn(t_ids, valid, x_sorted, W1, b1, W2, b2):
  return pl.pallas_call(
      _ffn_body,
      grid_spec=pltpu.PrefetchScalarGridSpec(
          num_scalar_prefetch=2,
          grid=(G,),
          in_specs=[
              pl.BlockSpec((T, D), lambda t, s, v: (t, 0)),
              pl.BlockSpec((1, H, D), lambda t, s, v: (s[t], 0, 0)),
              pl.BlockSpec((1, 1, H), lambda t, s, v: (s[t], 0, 0)),
              pl.BlockSpec((1, H, H), lambda t, s, v: (s[t], 0, 0)),
              pl.BlockSpec((1, 1, H), lambda t, s, v: (s[t], 0, 0)),
          ],
          out_specs=pl.BlockSpec((T, H), lambda t, s, v: (t, 0)),
      ),
      out_shape=jax.ShapeDtypeStruct((N_PAD, H), jnp.float32),
  )(t_ids, valid, x_sorted, W1, b1, W2, b2)


def kernel(x, Wr, br, W1, b1, W2, b2):
  x_fp32 = x.astype(jnp.float32)
  routing_logits = x_fp32 @ Wr.T + br
  expert_ids = jnp.argmax(routing_logits, axis=1)

  onehot = (expert_ids[:, None] == jnp.arange(E, dtype=expert_ids.dtype)
            ).astype(jnp.int32)
  csum = jnp.cumsum(onehot, axis=0)
  counts = csum[-1]
  tiles_e = (counts + T - 1) // T
  tile_end = jnp.cumsum(tiles_e)
  row_start = (tile_end - tiles_e) * T
  slot = jnp.sum((csum + row_start[None, :] - 1) * onehot, axis=1)
  tile_idx = jnp.arange(G, dtype=jnp.int32)
  raw = jnp.sum((tile_idx[:, None] >= tile_end[None, :]).astype(jnp.int32),
                axis=1)
  valid = (tile_idx < tile_end[-1]).astype(jnp.int32)
  last_e = jnp.max(jnp.where(counts > 0, jnp.arange(E), 0)).astype(jnp.int32)
  t_ids = jnp.where(valid == 1, raw, last_e).astype(jnp.int32)

  slot3 = slot.astype(jnp.int32).reshape(_NW, (N // _NW) // _CHUNK, _CHUNK)
  x_sorted = _sc_scatter_rows(D, N, N_PAD)(x, slot3)

  out_sorted = _grouped_ffn(t_ids, valid, x_sorted, W1, b1[:, None, :], W2,
                            b2[:, None, :])

  outputs = _sc_gather_rows(N_PAD, H, N)(out_sorted, slot.astype(jnp.int32))
  return outputs, expert_ids
